# baseline (speedup 1.0000x reference)
import jax
import jax.numpy as jnp
from jax.experimental import pallas as pl
from jax.experimental.pallas import tpu as pltpu

_B, _C, _F, _H, _W = 8, 3, 64, 128, 128
_BC = _B * _C
_HW = _H * _W
_KEEP = 32


def _score(pooled, x_conv, fc_wT_ref, fc_b_ref, M1_ref, b1_ref, GG_ref,
           Mr_ref, br_ref, Ml_ref, bl_ref, gamr_ref, betr_ref, A8_ref,
           B8_ref, P_ref, Q_ref, a_ref):
    hi = jax.lax.Precision.HIGHEST
    bf = jnp.bfloat16
    f32 = jnp.float32

    def dot(a, b):
        return jax.lax.dot(a, b, precision=hi)

    def dotb(a, b):
        return jax.lax.dot(a.astype(bf), b.astype(bf),
                           preferred_element_type=f32)

    scores = jax.nn.sigmoid(dotb(x_conv, fc_wT_ref[...]) + fc_b_ref[...])

    r8 = dot(pooled, A8_ref[...])
    c8 = dot(pooled, B8_ref[...])
    xr0 = dotb(M1_ref[...], r8) + b1_ref[...]
    xc0 = dotb(M1_ref[...], c8) + b1_ref[...]

    rs = jnp.sum(xr0, axis=1, keepdims=True) + jnp.sum(xc0, axis=1, keepdims=True)
    mur = dot(GG_ref[...], rs) * (1.0 / 128.0)
    dr = xr0 - mur
    dc = xc0 - mur
    rs2 = (jnp.sum(dr * dr, axis=1, keepdims=True)
           + jnp.sum(dc * dc, axis=1, keepdims=True))
    varr = dot(GG_ref[...], rs2) * (1.0 / 128.0)
    inv = gamr_ref[...] / jnp.sqrt(varr + 1e-5)
    sr = jax.nn.sigmoid(dr * inv + betr_ref[...])
    sc = jax.nn.sigmoid(dc * inv + betr_ref[...])

    ar = jax.nn.sigmoid(dotb(Mr_ref[...], sr) + br_ref[...])
    al = jax.nn.sigmoid(dotb(Ml_ref[...], sc) + bl_ref[...])
    att = dot(ar, P_ref[...]) * dot(al, Q_ref[...])

    a = a_ref[0, 0]
    fin = a * att + (1.0 - a) * scores

    ff = fin[:, :, None]
    fg = fin[:, None, :]
    io_f = jax.lax.broadcasted_iota(jnp.int32, (_BC, _F, _F), 1)
    io_g = jax.lax.broadcasted_iota(jnp.int32, (_BC, _F, _F), 2)
    beats = (fg > ff) | ((fg == ff) & (io_g < io_f))
    cnt = jnp.sum(beats.astype(jnp.float32), axis=2)
    return (cnt < float(_KEEP)).astype(jnp.float32)


def _pool_score_body(x_ref, cw_ref, conv_b_ref, fc_wT_ref, fc_b_ref, M1_ref,
                     b1_ref, GG_ref, Mr_ref, br_ref, Ml_ref, bl_ref, gamr_ref,
                     betr_ref, A8_ref, B8_ref, P_ref, Q_ref, a_ref,
                     pooled_ref, xconv_ref, mask_ref):
    b = pl.program_id(0)
    c = pl.program_id(1)
    i = b * _C + c
    x = x_ref[0, 0].reshape(_F, _HW)
    s = jnp.sum(x, axis=-1) * (1.0 / _HW)
    prod = jax.lax.dot(cw_ref[...].astype(jnp.bfloat16),
                       x.astype(jnp.bfloat16),
                       preferred_element_type=jnp.float32)
    sc = jnp.sum(prod, axis=-1) * (1.0 / _HW)
    pooled_ref[pl.ds(i, 1), :] = s.reshape(1, _F)
    xconv_ref[pl.ds(i, 1), :] = sc.reshape(1, _F)

    @pl.when(i == _BC - 1)
    def _():
        x_conv = xconv_ref[...] + conv_b_ref[...]
        mask_ref[...] = _score(pooled_ref[...], x_conv, fc_wT_ref, fc_b_ref,
                               M1_ref, b1_ref, GG_ref, Mr_ref, br_ref, Ml_ref,
                               bl_ref, gamr_ref, betr_ref, A8_ref, B8_ref,
                               P_ref, Q_ref, a_ref)


def _mul_body(idx_ref, x_ref, o1_ref, o2_ref):
    b = pl.program_id(0)
    c = pl.program_id(1)
    f = pl.program_id(2)
    keep = idx_ref[(b * _C + c) * _F + f] == f
    m = jnp.where(keep, 1.0, 0.0).astype(jnp.float32)
    o1_ref[...] = x_ref[...] * m
    o2_ref[...] = jnp.zeros_like(o2_ref)


def kernel(x_freq, conv_w, conv_b, conv1_w, conv1_b, convr_w, convr_b,
           convl_w, convl_b, bn_gamma, bn_beta, fc_w, fc_b, a_param):
    f32 = jnp.float32

    eyeB = jnp.eye(_B, dtype=f32)
    M1 = jnp.kron(eyeB, conv1_w)
    Mr = jnp.kron(eyeB, convr_w)
    Ml = jnp.kron(eyeB, convl_w)
    b1 = jnp.tile(conv1_b, _B).reshape(_BC, 1)
    br = jnp.tile(convr_b, _B).reshape(_BC, 1)
    bl = jnp.tile(convl_b, _B).reshape(_BC, 1)
    gamr = jnp.tile(bn_gamma, _B).reshape(_BC, 1)
    betr = jnp.tile(bn_beta, _B).reshape(_BC, 1)
    ch = jnp.arange(_BC) % _C
    GG = (ch[:, None] == ch[None, :]).astype(f32)
    q8 = jnp.arange(_F, dtype=jnp.int32)
    A8 = ((q8[:, None] // 8) == jnp.arange(8)[None, :]).astype(f32) / 8.0
    B8 = ((q8[:, None] % 8) == jnp.arange(8)[None, :]).astype(f32) / 8.0
    P = (jnp.arange(8)[:, None] == (q8[None, :] // 8)).astype(f32)
    Q = (jnp.arange(8)[:, None] == (q8[None, :] % 8)).astype(f32)

    small = lambda a: pl.BlockSpec(a.shape, lambda b, c: (0,) * a.ndim)
    smalls = [conv_b.reshape(1, _F), fc_w.T, fc_b.reshape(1, _F), M1, b1, GG,
              Mr, br, Ml, bl, gamr, betr, A8, B8, P, Q,
              jnp.asarray(a_param, f32).reshape(1, 1)]

    _, _, mask = pl.pallas_call(
        _pool_score_body,
        grid=(_B, _C),
        in_specs=[pl.BlockSpec((1, 1, _F, _H, _W), lambda b, c: (b, c, 0, 0, 0)),
                  pl.BlockSpec((_F, _F), lambda b, c: (0, 0))]
                 + [small(a) for a in smalls],
        out_specs=[pl.BlockSpec((_BC, _F), lambda b, c: (0, 0)),
                   pl.BlockSpec((_BC, _F), lambda b, c: (0, 0)),
                   pl.BlockSpec((_BC, _F), lambda b, c: (0, 0))],
        out_shape=[jax.ShapeDtypeStruct((_BC, _F), f32),
                   jax.ShapeDtypeStruct((_BC, _F), f32),
                   jax.ShapeDtypeStruct((_BC, _F), f32)],
    )(x_freq, conv_w, *smalls)

    idxs = jnp.arange(_F, dtype=jnp.int32)
    cand = jnp.where(mask > 0.5, idxs[None, :], _F)
    nxt = jnp.flip(jax.lax.cummin(jnp.flip(cand, axis=1), axis=1), axis=1)
    last = jnp.max(jnp.where(mask > 0.5, idxs[None, :], -1), axis=1,
                   keepdims=True)
    red = jnp.where(nxt == _F, last, nxt).astype(jnp.int32).reshape(-1)

    shape5 = (_B, _C, _F, _H, _W)
    blk1 = (1, 1, 1, _H, _W)
    grid_spec = pltpu.PrefetchScalarGridSpec(
        num_scalar_prefetch=1,
        grid=(_B, _C, _F),
        in_specs=[pl.BlockSpec(
            blk1, lambda b, c, f, idx: (b, c, idx[(b * _C + c) * _F + f], 0, 0))],
        out_specs=[pl.BlockSpec(blk1, lambda b, c, f, idx: (b, c, f, 0, 0)),
                   pl.BlockSpec(blk1, lambda b, c, f, idx: (b, c, f, 0, 0))],
    )
    out1, out2 = pl.pallas_call(
        _mul_body,
        grid_spec=grid_spec,
        out_shape=[jax.ShapeDtypeStruct(shape5, f32),
                   jax.ShapeDtypeStruct(shape5, f32)],
    )(red, x_freq)

    return (out1, out2)


# speedup vs baseline: 5.0381x; 5.0381x over previous
import jax
import jax.numpy as jnp
from jax.experimental import pallas as pl
from jax.experimental.pallas import tpu as pltpu

_B, _C, _F, _H, _W = 8, 3, 64, 128, 128
_BC = _B * _C
_HW = _H * _W
_KEEP = 32


def _score(pooled, x_conv, fc_wT_ref, fc_b_ref, M1_ref, b1_ref, GG_ref,
           Mr_ref, br_ref, Ml_ref, bl_ref, gamr_ref, betr_ref, A8_ref,
           B8_ref, P_ref, Q_ref, a_ref):
    hi = jax.lax.Precision.HIGHEST
    bf = jnp.bfloat16
    f32 = jnp.float32

    def dot(a, b):
        return jax.lax.dot(a, b, precision=hi)

    def dotb(a, b):
        return jax.lax.dot(a.astype(bf), b.astype(bf),
                           preferred_element_type=f32)

    scores = jax.nn.sigmoid(dotb(x_conv, fc_wT_ref[...]) + fc_b_ref[...])

    r8 = dot(pooled, A8_ref[...])
    c8 = dot(pooled, B8_ref[...])
    xr0 = dotb(M1_ref[...], r8) + b1_ref[...]
    xc0 = dotb(M1_ref[...], c8) + b1_ref[...]

    rs = jnp.sum(xr0, axis=1, keepdims=True) + jnp.sum(xc0, axis=1, keepdims=True)
    mur = dot(GG_ref[...], rs) * (1.0 / 128.0)
    dr = xr0 - mur
    dc = xc0 - mur
    rs2 = (jnp.sum(dr * dr, axis=1, keepdims=True)
           + jnp.sum(dc * dc, axis=1, keepdims=True))
    varr = dot(GG_ref[...], rs2) * (1.0 / 128.0)
    inv = gamr_ref[...] / jnp.sqrt(varr + 1e-5)
    sr = jax.nn.sigmoid(dr * inv + betr_ref[...])
    sc = jax.nn.sigmoid(dc * inv + betr_ref[...])

    ar = jax.nn.sigmoid(dotb(Mr_ref[...], sr) + br_ref[...])
    al = jax.nn.sigmoid(dotb(Ml_ref[...], sc) + bl_ref[...])
    att = dot(ar, P_ref[...]) * dot(al, Q_ref[...])

    a = a_ref[0, 0]
    fin = a * att + (1.0 - a) * scores

    ff = fin[:, :, None]
    fg = fin[:, None, :]
    io_f = jax.lax.broadcasted_iota(jnp.int32, (_BC, _F, _F), 1)
    io_g = jax.lax.broadcasted_iota(jnp.int32, (_BC, _F, _F), 2)
    beats = (fg > ff) | ((fg == ff) & (io_g < io_f))
    cnt = jnp.sum(beats.astype(jnp.float32), axis=2)
    return (cnt < float(_KEEP)).astype(jnp.float32)


def _pool_score_body(x_ref, cw_ref, conv_b_ref, fc_wT_ref, fc_b_ref, M1_ref,
                     b1_ref, GG_ref, Mr_ref, br_ref, Ml_ref, bl_ref, gamr_ref,
                     betr_ref, A8_ref, B8_ref, P_ref, Q_ref, a_ref,
                     pooled_ref, xconv_ref, mask_ref, zeros_ref):
    b = pl.program_id(0)
    c = pl.program_id(1)
    i = b * _C + c
    zeros_ref[...] = jnp.zeros_like(zeros_ref)
    x = x_ref[0, 0].reshape(_F, _HW)
    s = jnp.sum(x, axis=-1) * (1.0 / _HW)
    prod = jax.lax.dot(cw_ref[...].astype(jnp.bfloat16),
                       x.astype(jnp.bfloat16),
                       preferred_element_type=jnp.float32)
    sc = jnp.sum(prod, axis=-1) * (1.0 / _HW)
    pooled_ref[pl.ds(i, 1), :] = s.reshape(1, _F)
    xconv_ref[pl.ds(i, 1), :] = sc.reshape(1, _F)

    @pl.when(i == _BC - 1)
    def _():
        x_conv = xconv_ref[...] + conv_b_ref[...]
        mask_ref[...] = _score(pooled_ref[...], x_conv, fc_wT_ref, fc_b_ref,
                               M1_ref, b1_ref, GG_ref, Mr_ref, br_ref, Ml_ref,
                               bl_ref, gamr_ref, betr_ref, A8_ref, B8_ref,
                               P_ref, Q_ref, a_ref)


def _mul_body(x_ref, m_ref, o1_ref):
    o1_ref[...] = x_ref[...] * m_ref[...]


def kernel(x_freq, conv_w, conv_b, conv1_w, conv1_b, convr_w, convr_b,
           convl_w, convl_b, bn_gamma, bn_beta, fc_w, fc_b, a_param):
    f32 = jnp.float32

    eyeB = jnp.eye(_B, dtype=f32)
    M1 = jnp.kron(eyeB, conv1_w)
    Mr = jnp.kron(eyeB, convr_w)
    Ml = jnp.kron(eyeB, convl_w)
    b1 = jnp.tile(conv1_b, _B).reshape(_BC, 1)
    br = jnp.tile(convr_b, _B).reshape(_BC, 1)
    bl = jnp.tile(convl_b, _B).reshape(_BC, 1)
    gamr = jnp.tile(bn_gamma, _B).reshape(_BC, 1)
    betr = jnp.tile(bn_beta, _B).reshape(_BC, 1)
    ch = jnp.arange(_BC) % _C
    GG = (ch[:, None] == ch[None, :]).astype(f32)
    q8 = jnp.arange(_F, dtype=jnp.int32)
    A8 = ((q8[:, None] // 8) == jnp.arange(8)[None, :]).astype(f32) / 8.0
    B8 = ((q8[:, None] % 8) == jnp.arange(8)[None, :]).astype(f32) / 8.0
    P = (jnp.arange(8)[:, None] == (q8[None, :] // 8)).astype(f32)
    Q = (jnp.arange(8)[:, None] == (q8[None, :] % 8)).astype(f32)

    small = lambda a: pl.BlockSpec(a.shape, lambda b, c: (0,) * a.ndim)
    smalls = [conv_b.reshape(1, _F), fc_w.T, fc_b.reshape(1, _F), M1, b1, GG,
              Mr, br, Ml, bl, gamr, betr, A8, B8, P, Q,
              jnp.asarray(a_param, f32).reshape(1, 1)]

    shape5 = (_B, _C, _F, _H, _W)
    blk5 = (1, 1, _F, _H, _W)
    _, _, mask, out2 = pl.pallas_call(
        _pool_score_body,
        grid=(_B, _C),
        in_specs=[pl.BlockSpec((1, 1, _F, _H, _W), lambda b, c: (b, c, 0, 0, 0)),
                  pl.BlockSpec((_F, _F), lambda b, c: (0, 0))]
                 + [small(a) for a in smalls],
        out_specs=[pl.BlockSpec((_BC, _F), lambda b, c: (0, 0)),
                   pl.BlockSpec((_BC, _F), lambda b, c: (0, 0)),
                   pl.BlockSpec((_BC, _F), lambda b, c: (0, 0)),
                   pl.BlockSpec(blk5, lambda b, c: (b, c, 0, 0, 0))],
        out_shape=[jax.ShapeDtypeStruct((_BC, _F), f32),
                   jax.ShapeDtypeStruct((_BC, _F), f32),
                   jax.ShapeDtypeStruct((_BC, _F), f32),
                   jax.ShapeDtypeStruct(shape5, f32)],
    )(x_freq, conv_w, *smalls)

    mask5 = mask.reshape(_B, _C, _F, 1, 1)
    out1 = pl.pallas_call(
        _mul_body,
        grid=(_B, _C),
        in_specs=[pl.BlockSpec(blk5, lambda b, c: (b, c, 0, 0, 0)),
                  pl.BlockSpec((1, 1, _F, 1, 1), lambda b, c: (b, c, 0, 0, 0))],
        out_specs=pl.BlockSpec(blk5, lambda b, c: (b, c, 0, 0, 0)),
        out_shape=jax.ShapeDtypeStruct(shape5, f32),
    )(x_freq, mask5)

    return (out1, out2)


# speedup vs baseline: 5.0613x; 1.0046x over previous
import jax
import jax.numpy as jnp
from jax.experimental import pallas as pl
from jax.experimental.pallas import tpu as pltpu

_B, _C, _F, _H, _W = 8, 3, 64, 128, 128
_BC = _B * _C
_HW = _H * _W
_KEEP = 32


def _score(pooled, x_conv, fc_wT_ref, fc_b_ref, M1_ref, b1_ref, GG_ref,
           Mr_ref, br_ref, Ml_ref, bl_ref, gamr_ref, betr_ref, A8_ref,
           B8_ref, P_ref, Q_ref, a_ref):
    hi = jax.lax.Precision.HIGHEST
    bf = jnp.bfloat16
    f32 = jnp.float32

    def dot(a, b):
        return jax.lax.dot(a, b, precision=hi)

    def dotb(a, b):
        return jax.lax.dot(a.astype(bf), b.astype(bf),
                           preferred_element_type=f32)

    scores = jax.nn.sigmoid(dotb(x_conv, fc_wT_ref[...]) + fc_b_ref[...])

    r8 = dot(pooled, A8_ref[...])
    c8 = dot(pooled, B8_ref[...])
    xr0 = dotb(M1_ref[...], r8) + b1_ref[...]
    xc0 = dotb(M1_ref[...], c8) + b1_ref[...]

    rs = jnp.sum(xr0, axis=1, keepdims=True) + jnp.sum(xc0, axis=1, keepdims=True)
    mur = dot(GG_ref[...], rs) * (1.0 / 128.0)
    dr = xr0 - mur
    dc = xc0 - mur
    rs2 = (jnp.sum(dr * dr, axis=1, keepdims=True)
           + jnp.sum(dc * dc, axis=1, keepdims=True))
    varr = dot(GG_ref[...], rs2) * (1.0 / 128.0)
    inv = gamr_ref[...] / jnp.sqrt(varr + 1e-5)
    sr = jax.nn.sigmoid(dr * inv + betr_ref[...])
    sc = jax.nn.sigmoid(dc * inv + betr_ref[...])

    ar = jax.nn.sigmoid(dotb(Mr_ref[...], sr) + br_ref[...])
    al = jax.nn.sigmoid(dotb(Ml_ref[...], sc) + bl_ref[...])
    att = dot(ar, P_ref[...]) * dot(al, Q_ref[...])

    a = a_ref[0, 0]
    fin = a * att + (1.0 - a) * scores

    ff = fin[:, :, None]
    fg = fin[:, None, :]
    io_f = jax.lax.broadcasted_iota(jnp.int32, (_BC, _F, _F), 1)
    io_g = jax.lax.broadcasted_iota(jnp.int32, (_BC, _F, _F), 2)
    beats = (fg > ff) | ((fg == ff) & (io_g < io_f))
    cnt = jnp.sum(beats.astype(jnp.float32), axis=2)
    return (cnt < float(_KEEP)).astype(jnp.float32)


def _fused_body(x_ref, cw_ref, conv_b_ref, fc_wT_ref, fc_b_ref, M1_ref,
                b1_ref, GG_ref, Mr_ref, br_ref, Ml_ref, bl_ref, gamr_ref,
                betr_ref, A8_ref, B8_ref, P_ref, Q_ref, a_ref,
                o1_ref, o2_ref, pooled_s, xconv_s, maskb_s):
    ph = pl.program_id(0)
    b = pl.program_id(1)
    c = pl.program_id(2)
    i = b * _C + c

    @pl.when(ph == 0)
    def _pool_phase():
        o2_ref[...] = jnp.zeros_like(o2_ref)
        x = x_ref[0, 0].reshape(_F, _HW)
        s = jnp.sum(x, axis=-1) * (1.0 / _HW)
        prod = jax.lax.dot(cw_ref[...].astype(jnp.bfloat16),
                           x.astype(jnp.bfloat16),
                           preferred_element_type=jnp.float32)
        sc = jnp.sum(prod, axis=-1) * (1.0 / _HW)
        pooled_s[pl.ds(i, 1), :] = s.reshape(1, _F)
        xconv_s[pl.ds(i, 1), :] = sc.reshape(1, _F)

        @pl.when(i == _BC - 1)
        def _score_step():
            x_conv = xconv_s[...] + conv_b_ref[...]
            mask = _score(pooled_s[...], x_conv, fc_wT_ref, fc_b_ref,
                          M1_ref, b1_ref, GG_ref, Mr_ref, br_ref, Ml_ref,
                          bl_ref, gamr_ref, betr_ref, A8_ref, B8_ref,
                          P_ref, Q_ref, a_ref)
            maskt = mask.T
            maskb_s[...] = jnp.broadcast_to(maskt[:, :, None],
                                            (_F, _BC, 128))

    @pl.when(ph == 1)
    def _mul_phase():
        m = maskb_s[:, pl.ds(i, 1), :]
        o1_ref[0, 0] = x_ref[0, 0] * m


def kernel(x_freq, conv_w, conv_b, conv1_w, conv1_b, convr_w, convr_b,
           convl_w, convl_b, bn_gamma, bn_beta, fc_w, fc_b, a_param):
    f32 = jnp.float32

    eyeB = jnp.eye(_B, dtype=f32)
    M1 = jnp.kron(eyeB, conv1_w)
    Mr = jnp.kron(eyeB, convr_w)
    Ml = jnp.kron(eyeB, convl_w)
    b1 = jnp.tile(conv1_b, _B).reshape(_BC, 1)
    br = jnp.tile(convr_b, _B).reshape(_BC, 1)
    bl = jnp.tile(convl_b, _B).reshape(_BC, 1)
    gamr = jnp.tile(bn_gamma, _B).reshape(_BC, 1)
    betr = jnp.tile(bn_beta, _B).reshape(_BC, 1)
    ch = jnp.arange(_BC) % _C
    GG = (ch[:, None] == ch[None, :]).astype(f32)
    q8 = jnp.arange(_F, dtype=jnp.int32)
    A8 = ((q8[:, None] // 8) == jnp.arange(8)[None, :]).astype(f32) / 8.0
    B8 = ((q8[:, None] % 8) == jnp.arange(8)[None, :]).astype(f32) / 8.0
    P = (jnp.arange(8)[:, None] == (q8[None, :] // 8)).astype(f32)
    Q = (jnp.arange(8)[:, None] == (q8[None, :] % 8)).astype(f32)

    small = lambda a: pl.BlockSpec(a.shape, lambda ph, b, c: (0,) * a.ndim)
    smalls = [conv_b.reshape(1, _F), fc_w.T, fc_b.reshape(1, _F), M1, b1, GG,
              Mr, br, Ml, bl, gamr, betr, A8, B8, P, Q,
              jnp.asarray(a_param, f32).reshape(1, 1)]

    shape5 = (_B, _C, _F, _H, _W)
    blk5 = (1, 1, _F, _H, _W)

    def _o1_map(ph, b, c):
        z = jnp.int32(0)
        return (jnp.where(ph == 1, b, z), jnp.where(ph == 1, c, z), 0, 0, 0)

    def _o2_map(ph, b, c):
        z = jnp.int32(0)
        return (jnp.where(ph == 0, b, z), jnp.where(ph == 0, c, z), 0, 0, 0)

    out1, out2 = pl.pallas_call(
        _fused_body,
        grid=(2, _B, _C),
        in_specs=[pl.BlockSpec(blk5, lambda ph, b, c: (b, c, 0, 0, 0)),
                  pl.BlockSpec((_F, _F), lambda ph, b, c: (0, 0))]
                 + [small(a) for a in smalls],
        out_specs=[pl.BlockSpec(blk5, _o1_map),
                   pl.BlockSpec(blk5, _o2_map)],
        out_shape=[jax.ShapeDtypeStruct(shape5, f32),
                   jax.ShapeDtypeStruct(shape5, f32)],
        scratch_shapes=[pltpu.VMEM((_BC, _F), f32),
                        pltpu.VMEM((_BC, _F), f32),
                        pltpu.VMEM((_F, _BC, 128), f32)],
    )(x_freq, conv_w, *smalls)

    return (out1, out2)


# speedup vs baseline: 5.0764x; 1.0030x over previous
import jax
import jax.numpy as jnp
from jax.experimental import pallas as pl
from jax.experimental.pallas import tpu as pltpu

_B, _C, _F, _H, _W = 8, 3, 64, 128, 128
_BC = _B * _C
_HW = _H * _W
_KEEP = 32


def _score(pooled, x_conv, fc_wT_ref, fc_b_ref, M1_ref, b1_ref, GG_ref,
           Mr_ref, br_ref, Ml_ref, bl_ref, gamr_ref, betr_ref, AB_ref,
           a_ref):
    hi = jax.lax.Precision.HIGHEST
    bf = jnp.bfloat16
    f32 = jnp.float32

    def dot(a, b):
        return jax.lax.dot(a, b, precision=hi)

    def dotb(a, b):
        return jax.lax.dot(a.astype(bf), b.astype(bf),
                           preferred_element_type=f32)

    scores = jax.nn.sigmoid(dotb(x_conv, fc_wT_ref[...]) + fc_b_ref[...])

    rc = dot(pooled, AB_ref[...])
    x01 = dotb(M1_ref[...], rc)
    xr0 = x01[:, :8] + b1_ref[...]
    xc0 = x01[:, 8:] + b1_ref[...]

    rs = jnp.sum(xr0, axis=1, keepdims=True) + jnp.sum(xc0, axis=1, keepdims=True)
    mur = dot(GG_ref[...], rs) * (1.0 / 128.0)
    dr = xr0 - mur
    dc = xc0 - mur
    rs2 = (jnp.sum(dr * dr, axis=1, keepdims=True)
           + jnp.sum(dc * dc, axis=1, keepdims=True))
    varr = dot(GG_ref[...], rs2) * (1.0 / 128.0)
    inv = gamr_ref[...] / jnp.sqrt(varr + 1e-5)
    sr = jax.nn.sigmoid(dr * inv + betr_ref[...])
    sc = jax.nn.sigmoid(dc * inv + betr_ref[...])

    ar = jax.nn.sigmoid(dotb(Mr_ref[...], sr) + br_ref[...])
    al = jax.nn.sigmoid(dotb(Ml_ref[...], sc) + bl_ref[...])
    arP = jnp.broadcast_to(ar[:, :, None], (_BC, 8, 8)).reshape(_BC, _F)
    alQ = jnp.broadcast_to(al[:, None, :], (_BC, 8, 8)).reshape(_BC, _F)
    att = arP * alQ

    a = a_ref[0, 0]
    fin = a * att + (1.0 - a) * scores

    ff = fin[:, :, None]
    fg = fin[:, None, :]
    io_f = jax.lax.broadcasted_iota(jnp.int32, (_BC, _F, _F), 1)
    io_g = jax.lax.broadcasted_iota(jnp.int32, (_BC, _F, _F), 2)
    beats = (fg > ff) | ((fg == ff) & (io_g < io_f))
    cnt = jnp.sum(beats.astype(jnp.float32), axis=2)
    return (cnt < float(_KEEP)).astype(jnp.float32)


def _fused_body(x_ref, cw_ref, conv_b_ref, fc_wT_ref, fc_b_ref, M1_ref,
                b1_ref, GG_ref, Mr_ref, br_ref, Ml_ref, bl_ref, gamr_ref,
                betr_ref, AB_ref, a_ref,
                o1_ref, o2_ref, pooled_s, xconv_s, maskb_s):
    ph = pl.program_id(0)
    b = pl.program_id(1)
    c = pl.program_id(2)
    i = b * _C + c

    @pl.when(ph == 0)
    def _pool_phase():
        o2_ref[...] = jnp.zeros_like(o2_ref)
        x = x_ref[0, 0].reshape(_F, _HW)
        s = jnp.sum(x, axis=-1) * (1.0 / _HW)
        prod = jax.lax.dot(cw_ref[...].astype(jnp.bfloat16),
                           x.astype(jnp.bfloat16),
                           preferred_element_type=jnp.float32)
        sc = jnp.sum(prod, axis=-1) * (1.0 / _HW)
        pooled_s[pl.ds(i, 1), :] = s.reshape(1, _F)
        xconv_s[pl.ds(i, 1), :] = sc.reshape(1, _F)

        @pl.when(i == _BC - 1)
        def _score_step():
            x_conv = xconv_s[...] + conv_b_ref[...]
            mask = _score(pooled_s[...], x_conv, fc_wT_ref, fc_b_ref,
                          M1_ref, b1_ref, GG_ref, Mr_ref, br_ref, Ml_ref,
                          bl_ref, gamr_ref, betr_ref, AB_ref, a_ref)
            maskt = mask.T
            maskb_s[...] = jnp.broadcast_to(maskt[:, :, None],
                                            (_F, _BC, 128))

    @pl.when(ph == 1)
    def _mul_phase():
        m = maskb_s[:, pl.ds(i, 1), :]
        o1_ref[0, 0] = x_ref[0, 0] * m


def kernel(x_freq, conv_w, conv_b, conv1_w, conv1_b, convr_w, convr_b,
           convl_w, convl_b, bn_gamma, bn_beta, fc_w, fc_b, a_param):
    f32 = jnp.float32

    eyeB = jnp.eye(_B, dtype=f32)
    M1 = jnp.kron(eyeB, conv1_w)
    Mr = jnp.kron(eyeB, convr_w)
    Ml = jnp.kron(eyeB, convl_w)
    b1 = jnp.tile(conv1_b, _B).reshape(_BC, 1)
    br = jnp.tile(convr_b, _B).reshape(_BC, 1)
    bl = jnp.tile(convl_b, _B).reshape(_BC, 1)
    gamr = jnp.tile(bn_gamma, _B).reshape(_BC, 1)
    betr = jnp.tile(bn_beta, _B).reshape(_BC, 1)
    ch = jnp.arange(_BC) % _C
    GG = (ch[:, None] == ch[None, :]).astype(f32)
    q8 = jnp.arange(_F, dtype=jnp.int32)
    A8 = ((q8[:, None] // 8) == jnp.arange(8)[None, :]).astype(f32) / 8.0
    B8 = ((q8[:, None] % 8) == jnp.arange(8)[None, :]).astype(f32) / 8.0
    AB = jnp.concatenate([A8, B8], axis=1)

    small = lambda a: pl.BlockSpec(a.shape, lambda ph, b, c: (0,) * a.ndim)
    smalls = [conv_b.reshape(1, _F), fc_w.T, fc_b.reshape(1, _F), M1, b1, GG,
              Mr, br, Ml, bl, gamr, betr, AB,
              jnp.asarray(a_param, f32).reshape(1, 1)]

    shape5 = (_B, _C, _F, _H, _W)
    blk5 = (1, 1, _F, _H, _W)

    def _o1_map(ph, b, c):
        z = jnp.int32(0)
        return (jnp.where(ph == 1, b, z), jnp.where(ph == 1, c, z), 0, 0, 0)

    def _o2_map(ph, b, c):
        z = jnp.int32(0)
        return (jnp.where(ph == 0, b, z), jnp.where(ph == 0, c, z), 0, 0, 0)

    out1, out2 = pl.pallas_call(
        _fused_body,
        grid=(2, _B, _C),
        in_specs=[pl.BlockSpec(blk5, lambda ph, b, c: (b, c, 0, 0, 0)),
                  pl.BlockSpec((_F, _F), lambda ph, b, c: (0, 0))]
                 + [small(a) for a in smalls],
        out_specs=[pl.BlockSpec(blk5, _o1_map),
                   pl.BlockSpec(blk5, _o2_map)],
        out_shape=[jax.ShapeDtypeStruct(shape5, f32),
                   jax.ShapeDtypeStruct(shape5, f32)],
        scratch_shapes=[pltpu.VMEM((_BC, _F), f32),
                        pltpu.VMEM((_BC, _F), f32),
                        pltpu.VMEM((_F, _BC, 128), f32)],
    )(x_freq, conv_w, *smalls)

    return (out1, out2)
